# baseline (speedup 1.0000x reference)
import functools

import jax
import jax.numpy as jnp
from jax import lax
from jax.experimental import pallas as pl
from jax.experimental.pallas import tpu as pltpu
from jax.experimental.pallas import tpu_sc as plsc

_LANES = 16


def _sc_stage(inputs_f, predict_f, normal_f, I_H, H_U, B, K, D, NC, NS):
    NW = NC * NS
    BW = B // NW
    CHUNK = 128
    CPW = (BW * K) // CHUNK
    mesh = plsc.VectorSubcoreMesh(core_axis_name="c", subcore_axis_name="s")

    @functools.partial(
        pl.kernel,
        mesh=mesh,
        out_type=(
            jax.ShapeDtypeStruct((B,), jnp.float32),
            jax.ShapeDtypeStruct((B,), jnp.float32),
        ),
        scratch_types=[
            pltpu.VMEM((BW,), jnp.int32),
            pltpu.VMEM((BW,), jnp.int32),
            pltpu.VMEM((BW * K,), jnp.int32),
            pltpu.VMEM((BW, D), jnp.float32),
            pltpu.VMEM((BW, D), jnp.float32),
            pltpu.VMEM((BW * K, D), jnp.float32),
            pltpu.VMEM((BW,), jnp.float32),
            pltpu.VMEM((BW,), jnp.float32),
            pltpu.SemaphoreType.DMA,
        ],
        compiler_params=pltpu.CompilerParams(
            needs_layout_passes=False, use_tc_tiling_on_sc=False),
    )
    def sc_kernel(inputs_hbm, predict_hbm, normal_hbm, ih_hbm, hu_hbm,
                  denom_hbm, scores_hbm,
                  iidx_v, pidx_v, nidx_v, irows_v, prows_v, nrows_v,
                  denom_v, scores_v, sem):
        wid = lax.axis_index("s") * NC + lax.axis_index("c")
        base = wid * BW
        pltpu.sync_copy(inputs_hbm.at[pl.ds(base, BW)], iidx_v)
        pltpu.sync_copy(predict_hbm.at[pl.ds(base, BW)], pidx_v)
        pltpu.sync_copy(normal_hbm.at[pl.ds(wid * BW * K, BW * K)], nidx_v)
        copies = [
            pltpu.async_copy(ih_hbm.at[iidx_v], irows_v, sem),
            pltpu.async_copy(hu_hbm.at[pidx_v], prows_v, sem),
        ]
        for j in range(CPW):
            copies.append(pltpu.async_copy(
                hu_hbm.at[nidx_v.at[pl.ds(j * CHUNK, CHUNK)]],
                nrows_v.at[pl.ds(j * CHUNK, CHUNK)], sem))
        for cp in copies:
            cp.wait()

        iota = lax.iota(jnp.int32, _LANES)
        cols = [jnp.full((_LANES,), d, jnp.int32) for d in range(D)]

        def blk(i, carry):
            b0 = i * _LANES
            bvec = b0 + iota
            icols = [plsc.load_gather(irows_v, [bvec, cols[d]])
                     for d in range(D)]
            dsum = jnp.zeros((_LANES,), jnp.float32)
            for k in range(K):
                rvec = bvec * K + k
                acc = jnp.zeros((_LANES,), jnp.float32)
                for d in range(D):
                    nv = plsc.load_gather(nrows_v, [rvec, cols[d]])
                    acc = acc + nv * icols[d]
                dsum = dsum + jnp.exp(acc)
            sc = jnp.zeros((_LANES,), jnp.float32)
            for d in range(D):
                pv = plsc.load_gather(prows_v, [bvec, cols[d]])
                sc = sc + pv * icols[d]
            denom_v[pl.ds(b0, _LANES)] = dsum
            scores_v[pl.ds(b0, _LANES)] = sc
            return carry

        lax.fori_loop(0, BW // _LANES, blk, 0)
        pltpu.sync_copy(denom_v, denom_hbm.at[pl.ds(base, BW)])
        pltpu.sync_copy(scores_v, scores_hbm.at[pl.ds(base, BW)])

    return sc_kernel(inputs_f, predict_f, normal_f, I_H, H_U)


def _tc_finish(denom, scores, B):
    def body(denom_ref, scores_ref, out_ref):
        dl = jnp.log(denom_ref[...])
        val = (jnp.sum(dl) - jnp.sum(scores_ref[...])) / B
        out_ref[...] = jnp.full((1, 1), val, jnp.float32)

    return pl.pallas_call(
        body,
        out_shape=jax.ShapeDtypeStruct((1, 1), jnp.float32),
    )(denom, scores)


def kernel(inputs, predict, normal, I_H, H_U):
    B = inputs.shape[0]
    K = normal.shape[1]
    D = I_H.shape[1]
    info = plsc.get_sparse_core_info()
    NC, NS = info.num_cores, info.num_subcores
    inputs_f = inputs.reshape(-1).astype(jnp.int32)
    predict_f = predict.reshape(-1).astype(jnp.int32)
    normal_f = normal.reshape(-1).astype(jnp.int32)
    denom, scores = _sc_stage(inputs_f, predict_f, normal_f, I_H, H_U,
                              B, K, D, NC, NS)
    nll = _tc_finish(denom.reshape(B // 128, 128), scores.reshape(B // 128, 128), B)
    return nll.reshape((1,))
